# baseline (speedup 1.0000x reference)
import functools

import jax
import jax.numpy as jnp
from jax import lax
from jax.experimental import pallas as pl
from jax.experimental.pallas import tpu as pltpu
from jax.experimental.pallas import tpu_sc as plsc

NUM_ITEMS = 1000000
NUM_USERS = 100000
FEAT = 64
B = 4096
SLATE = 20

_info = plsc.get_sparse_core_info()
NC, NS, L = _info.num_cores, _info.num_subcores, _info.num_lanes
NW = NC * NS
ROWS_PER_W = B // NW
CB = 64
NCH = ROWS_PER_W // CB
PAIRS = CB * SLATE
IDX_W = 128
NGATHER = PAIRS // IDX_W


def _body(s_hbm, u_hbm, doc_hbm, user_hbm, out_hbm,
          sidx, uidx, uerows, docrows, outv, semu, semd):
    wid = lax.axis_index("s") * NC + lax.axis_index("c")
    lane15 = lax.iota(jnp.int32, L) == (L - 1)
    lane_lo = lax.iota(jnp.int32, L) < (L // 2)
    idx_ev = (lax.iota(jnp.int32, L) % (L // 2)) * 2
    idx_od = idx_ev + 1
    for c in range(NCH):
        base = wid * ROWS_PER_W + c * CB
        pltpu.sync_copy(u_hbm.at[pl.ds(base, CB)], uidx)
        pltpu.sync_copy(s_hbm.at[wid * NCH + c], sidx)
        cp_u = pltpu.async_copy(user_hbm.at[uidx], uerows, semu)
        cp_d = [pltpu.async_copy(doc_hbm.at[sidx.at[j]],
                                 docrows.at[pl.ds(j * IDX_W, IDX_W)], semd)
                for j in range(NGATHER)]
        cp_u.wait()
        for cp in cp_d:
            cp.wait()

        def row_body(i, carry):
            u4 = [uerows[i, pl.ds(16 * k, 16)] for k in range(FEAT // L)]
            ue = []
            for k in range(FEAT // 32):
                lo, hi = u4[2 * k], u4[2 * k + 1]
                ue.append(jnp.where(lane_lo, jnp.take(lo, idx_ev),
                                    jnp.take(hi, idx_ev)))
                ue.append(jnp.where(lane_lo, jnp.take(lo, idx_od),
                                    jnp.take(hi, idx_od)))
            for l in range(SLATE):
                row = i * SLATE + l
                acc = None
                for k in range(FEAT // 32):
                    d32 = docrows[row, pl.ds(32 * k, 32)]
                    dev, dod = plsc.unpack(
                        d32, format=plsc.PackFormat.INTERLEAVED)
                    term = dev * ue[2 * k] + dod * ue[2 * k + 1]
                    acc = term if acc is None else acc + term
                csum = plsc.cumsum(acc)
                prob = 1.0 / (1.0 + jnp.exp(-csum))
                plsc.store_scatter(outv, [jnp.full((L,), row, jnp.int32)],
                                   prob, mask=lane15)
            return carry

        lax.fori_loop(0, CB, row_body, 0)
        pltpu.sync_copy(outv, out_hbm.at[pl.ds(base * SLATE, PAIRS)])


@jax.jit
def _run(s_flat2d, u, doc_table16, user_table):
    mesh = plsc.VectorSubcoreMesh(core_axis_name="c", subcore_axis_name="s")
    f = functools.partial(
        pl.kernel, mesh=mesh,
        out_type=jax.ShapeDtypeStruct((B * SLATE,), jnp.float32),
        compiler_params=pltpu.CompilerParams(needs_layout_passes=False,
                                             use_tc_tiling_on_sc=False),
        scratch_types=[
            pltpu.VMEM((NGATHER, IDX_W), jnp.int32),
            pltpu.VMEM((CB,), jnp.int32),
            pltpu.VMEM((CB, FEAT), jnp.float32),
            pltpu.VMEM((PAIRS, FEAT), jnp.bfloat16),
            pltpu.VMEM((PAIRS,), jnp.float32),
            pltpu.SemaphoreType.DMA,
            pltpu.SemaphoreType.DMA,
        ],
    )(_body)
    return f(s_flat2d, u, doc_table16, user_table)


def kernel(s, r, u, doc_table, user_table):
    del r
    s_flat2d = s.reshape(NW * NCH, NGATHER, IDX_W)
    doc16 = doc_table.astype(jnp.bfloat16)
    out = _run(s_flat2d, u, doc16, user_table)
    return out.reshape(B, SLATE)


# speedup vs baseline: 1.2842x; 1.2842x over previous
import functools

import jax
import jax.numpy as jnp
from jax import lax
from jax.experimental import pallas as pl
from jax.experimental.pallas import tpu as pltpu
from jax.experimental.pallas import tpu_sc as plsc

NUM_ITEMS = 1000000
NUM_USERS = 100000
FEAT = 64
B = 4096
SLATE = 20

_info = plsc.get_sparse_core_info()
NC, NS, L = _info.num_cores, _info.num_subcores, _info.num_lanes
NW = NC * NS
ROWS_PER_W = B // NW
CB = 64
NCH = ROWS_PER_W // CB
PAIRS = CB * SLATE
IDX_W = 128
NGATHER = PAIRS // IDX_W


def _body(s_hbm, u_hbm, doc_hbm, user_hbm, out_hbm,
          sidx, uidx, uerows, docrows, outv, semu, semd):
    wid = lax.axis_index("s") * NC + lax.axis_index("c")
    lane15 = lax.iota(jnp.int32, L) == (L - 1)
    for c in range(NCH):
        base = wid * ROWS_PER_W + c * CB
        pltpu.sync_copy(u_hbm.at[pl.ds(base, CB)], uidx)
        pltpu.sync_copy(s_hbm.at[wid * NCH + c], sidx)
        cp_u = pltpu.async_copy(user_hbm.at[uidx], uerows, semu)
        cp_d = [pltpu.async_copy(doc_hbm.at[sidx.at[j]],
                                 docrows.at[pl.ds(j * IDX_W, IDX_W)], semd)
                for j in range(NGATHER)]
        cp_u.wait()
        for cp in cp_d:
            cp.wait()

        def row_body(i, carry):
            ue = [uerows[i, pl.ds(16 * k, 16)] for k in range(FEAT // L)]
            for l in range(SLATE):
                row = i * SLATE + l
                acc = ue[0] * docrows[row, pl.ds(0, 16)]
                for k in range(1, FEAT // L):
                    acc = acc + ue[k] * docrows[row, pl.ds(16 * k, 16)]
                csum = plsc.cumsum(acc)
                prob = 1.0 / (1.0 + jnp.exp(-csum))
                plsc.store_scatter(outv, [jnp.full((L,), row, jnp.int32)],
                                   prob, mask=lane15)
            return carry

        lax.fori_loop(0, CB, row_body, 0)
        pltpu.sync_copy(outv, out_hbm.at[pl.ds(base * SLATE, PAIRS)])


@jax.jit
def _run(s_flat2d, u, doc_table, user_table):
    mesh = plsc.VectorSubcoreMesh(core_axis_name="c", subcore_axis_name="s")
    f = functools.partial(
        pl.kernel, mesh=mesh,
        out_type=jax.ShapeDtypeStruct((B * SLATE,), jnp.float32),
        compiler_params=pltpu.CompilerParams(needs_layout_passes=False,
                                             use_tc_tiling_on_sc=False),
        scratch_types=[
            pltpu.VMEM((NGATHER, IDX_W), jnp.int32),
            pltpu.VMEM((CB,), jnp.int32),
            pltpu.VMEM((CB, FEAT), jnp.float32),
            pltpu.VMEM((PAIRS, FEAT), jnp.float32),
            pltpu.VMEM((PAIRS,), jnp.float32),
            pltpu.SemaphoreType.DMA,
            pltpu.SemaphoreType.DMA,
        ],
    )(_body)
    return f(s_flat2d, u, doc_table, user_table)


def kernel(s, r, u, doc_table, user_table):
    del r
    s_flat2d = s.reshape(NW * NCH, NGATHER, IDX_W)
    out = _run(s_flat2d, u, doc_table, user_table)
    return out.reshape(B, SLATE)


# speedup vs baseline: 1.2913x; 1.0055x over previous
import functools

import jax
import jax.numpy as jnp
from jax import lax
from jax.experimental import pallas as pl
from jax.experimental.pallas import tpu as pltpu
from jax.experimental.pallas import tpu_sc as plsc

NUM_ITEMS = 1000000
NUM_USERS = 100000
FEAT = 64
B = 4096
SLATE = 20

_info = plsc.get_sparse_core_info()
NC, NS, L = _info.num_cores, _info.num_subcores, _info.num_lanes
NW = NC * NS
ROWS_PER_W = B // NW
CB = 32
NCH = ROWS_PER_W // CB
PAIRS = CB * SLATE
IDX_W = 128
NGATHER = PAIRS // IDX_W


def _body(s_hbm, u_hbm, doc_hbm, user_hbm, out_hbm,
          sidx, uidx, uerows, docrows, outv, semu0, semu1, semd0, semd1):
    wid = lax.axis_index("s") * NC + lax.axis_index("c")
    lane15 = lax.iota(jnp.int32, L) == (L - 1)
    semu = (semu0, semu1)
    semd = (semd0, semd1)

    def fetch(c):
        b = c % 2
        base = wid * ROWS_PER_W + c * CB
        pltpu.sync_copy(u_hbm.at[pl.ds(base, CB)], uidx.at[b])
        pltpu.sync_copy(s_hbm.at[wid * NCH + c], sidx.at[b])
        pltpu.async_copy(user_hbm.at[uidx.at[b]], uerows.at[b], semu[b])
        for j in range(NGATHER):
            pltpu.async_copy(doc_hbm.at[sidx.at[b, j]],
                             docrows.at[b, pl.ds(j * IDX_W, IDX_W)], semd[b])

    def drain(c):
        b = c % 2
        pltpu.make_async_copy(user_hbm.at[uidx.at[b]], uerows.at[b],
                              semu[b]).wait()
        for j in range(NGATHER):
            pltpu.make_async_copy(doc_hbm.at[sidx.at[b, j]],
                                  docrows.at[b, pl.ds(j * IDX_W, IDX_W)],
                                  semd[b]).wait()

    fetch(0)
    for c in range(NCH):
        b = c % 2
        if c + 1 < NCH:
            fetch(c + 1)
        drain(c)
        base = wid * ROWS_PER_W + c * CB

        def row_body(i, carry, b=b):
            ue = [uerows[b, i, pl.ds(16 * k, 16)] for k in range(FEAT // L)]
            for l in range(SLATE):
                row = i * SLATE + l
                acc = ue[0] * docrows[b, row, pl.ds(0, 16)]
                for k in range(1, FEAT // L):
                    acc = acc + ue[k] * docrows[b, row, pl.ds(16 * k, 16)]
                csum = plsc.cumsum(acc)
                prob = 1.0 / (1.0 + jnp.exp(-csum))
                plsc.store_scatter(outv, [jnp.full((L,), row, jnp.int32)],
                                   prob, mask=lane15)
            return carry

        lax.fori_loop(0, CB, row_body, 0)
        pltpu.sync_copy(outv, out_hbm.at[pl.ds(base * SLATE, PAIRS)])


@jax.jit
def _run(s_flat3d, u, doc_table, user_table):
    mesh = plsc.VectorSubcoreMesh(core_axis_name="c", subcore_axis_name="s")
    f = functools.partial(
        pl.kernel, mesh=mesh,
        out_type=jax.ShapeDtypeStruct((B * SLATE,), jnp.float32),
        compiler_params=pltpu.CompilerParams(needs_layout_passes=False,
                                             use_tc_tiling_on_sc=False),
        scratch_types=[
            pltpu.VMEM((2, NGATHER, IDX_W), jnp.int32),
            pltpu.VMEM((2, CB), jnp.int32),
            pltpu.VMEM((2, CB, FEAT), jnp.float32),
            pltpu.VMEM((2, PAIRS, FEAT), jnp.float32),
            pltpu.VMEM((PAIRS,), jnp.float32),
            pltpu.SemaphoreType.DMA,
            pltpu.SemaphoreType.DMA,
            pltpu.SemaphoreType.DMA,
            pltpu.SemaphoreType.DMA,
        ],
    )(_body)
    return f(s_flat3d, u, doc_table, user_table)


def kernel(s, r, u, doc_table, user_table):
    del r
    s_flat3d = s.reshape(NW * NCH, NGATHER, IDX_W)
    out = _run(s_flat3d, u, doc_table, user_table)
    return out.reshape(B, SLATE)
